# baseline (speedup 1.0000x reference)
import functools

import jax
import jax.numpy as jnp
from jax import lax
from jax.experimental import pallas as pl
from jax.experimental.pallas import tpu as pltpu
from jax.experimental.pallas import tpu_sc as plsc

D = 32
NC, NS = 2, 16
NW = NC * NS
CB = 128


def _body(n_chunks_per_w, chunks_per_s, n_ann,
          ann_rm, qtab, ai, qi, xl, feat5, param5,
          idxa_v, idxq_v, qtab_v, rows_v, feat_v, sem_g, sem_x):
    wid = lax.axis_index("s") * NC + lax.axis_index("c")

    pltpu.sync_copy(qtab, qtab_v)

    def chunk(t, carry):
        c = wid * n_chunks_per_w + t
        s = c // chunks_per_s
        tj = c % chunks_per_s
        j0 = c * CB
        b0 = tj * CB

        pltpu.sync_copy(ai.at[pl.ds(j0, CB)], idxa_v)
        pltpu.sync_copy(qi.at[pl.ds(j0, CB)], idxq_v)

        def remap(g, c2):
            v = idxa_v[pl.ds(g * 16, 16)]
            idxa_v[pl.ds(g * 16, 16)] = jnp.where(v < 0, n_ann, v)
            return c2
        lax.fori_loop(0, CB // 16, remap, 0)

        ca = pltpu.async_copy(ann_rm.at[idxa_v], rows_v, sem_g)
        cx = pltpu.async_copy(
            xl.at[pl.ds(1, D), s, pl.ds(b0, CB)],
            feat_v.at[pl.ds(D, D)], sem_x)
        ca.wait()

        def qadd(g, c2):
            bidx = lax.iota(jnp.int32, 16) + g * 16
            q16 = idxq_v[pl.ds(g * 16, 16)]
            qb = q16 * D
            for f in range(D):
                col = jnp.full((16,), f, jnp.int32)
                av = plsc.load_gather(rows_v, [bidx, col])
                qv = plsc.load_gather(qtab_v, [qb + f])
                feat_v[f, pl.ds(g * 16, 16)] = av + qv
            return c2
        lax.fori_loop(0, CB // 16, qadd, 0)

        cx.wait()
        for ti in range(2 * D // 8):
            pltpu.sync_copy(feat_v.at[pl.ds(ti * 8, 8)], feat5.at[s, ti, tj])
        for ti in range(D // 8):
            pltpu.sync_copy(feat_v.at[pl.ds(D + ti * 8, 8)],
                            param5.at[s, ti, tj])
        return carry

    lax.fori_loop(0, n_chunks_per_w, chunk, 0)


def kernel(x, annotators, questions, annotator_embedding, question_embedding):
    B, S, XF = x.shape
    N = B * S
    n_ann = annotator_embedding.shape[0] - 1
    assert B % CB == 0 and N % (NW * CB) == 0
    chunks_per_s = B // CB
    n_chunks_per_w = N // (NW * CB)

    ai = annotators.T.reshape(N).astype(jnp.int32)
    qi = questions.T.reshape(N).astype(jnp.int32)
    qtab = question_embedding.reshape(-1)
    xl = x.transpose(2, 1, 0)

    mesh = plsc.VectorSubcoreMesh(core_axis_name="c", subcore_axis_name="s")
    feat5, param5 = pl.kernel(
        functools.partial(_body, n_chunks_per_w, chunks_per_s, n_ann),
        out_type=(
            jax.ShapeDtypeStruct((S, 2 * D // 8, B // CB, 8, CB), jnp.float32),
            jax.ShapeDtypeStruct((S, D // 8, B // CB, 8, CB), jnp.float32),
        ),
        mesh=mesh,
        compiler_params=pltpu.CompilerParams(
            use_tc_tiling_on_sc=False, needs_layout_passes=False),
        scratch_types=[
            pltpu.VMEM((CB,), jnp.int32),
            pltpu.VMEM((CB,), jnp.int32),
            pltpu.VMEM((question_embedding.size,), jnp.float32),
            pltpu.VMEM((CB, D), jnp.float32),
            pltpu.VMEM((2 * D, CB), jnp.float32),
            pltpu.SemaphoreType.DMA,
            pltpu.SemaphoreType.DMA,
        ],
    )(annotator_embedding, qtab, ai, qi, xl)

    feature_x = feat5.transpose(2, 4, 0, 1, 3).reshape(B, S, 2 * D)
    param_x = param5.transpose(2, 4, 0, 1, 3).reshape(B, S, D)
    return feature_x, param_x


# speedup vs baseline: 1.2662x; 1.2662x over previous
import functools

import jax
import jax.numpy as jnp
from jax import lax
from jax.experimental import pallas as pl
from jax.experimental.pallas import tpu as pltpu
from jax.experimental.pallas import tpu_sc as plsc

D = 32
NC, NS = 2, 16
NW = NC * NS
CB = 128


FP = CB + 1


def _body(n_chunks_per_w, chunks_per_s, n_ann,
          ann_rm, qtab, ai, qi, xl, feat5, param5,
          idxa_v, idxq_v, qtab_v, rows_g, feat_v, x_v, sem_g, sem_x, sem_w):
    wid = lax.axis_index("s") * NC + lax.axis_index("c")

    pltpu.sync_copy(qtab, qtab_v)
    f16a = lax.iota(jnp.int32, 16)
    f16b = f16a + 16

    def chunk(t, carry):
        c = wid * n_chunks_per_w + t
        s = c // chunks_per_s
        tj = c % chunks_per_s
        j0 = c * CB
        b0 = tj * CB

        pltpu.sync_copy(ai.at[pl.ds(j0, CB)], idxa_v)
        pltpu.sync_copy(qi.at[pl.ds(j0, CB)], idxq_v.at[pl.ds(0, CB)])

        def remap(g, c2):
            v = idxa_v[pl.ds(g * 16, 16)]
            idxa_v[pl.ds(g * 16, 16)] = jnp.where(v < 0, n_ann, v)
            return c2
        lax.fori_loop(0, CB // 16, remap, 0)

        ca = pltpu.async_copy(ann_rm.at[idxa_v], rows_g, sem_g)
        cx = pltpu.async_copy(
            xl.at[pl.ds(1, D), s, pl.ds(b0, CB)], x_v, sem_x)
        ca.wait()

        def row(b, c2):
            qb = idxq_v[pl.ds(b, 16)][0] * D
            v0 = rows_g[b, pl.ds(0, 16)] + qtab_v[pl.ds(qb, 16)]
            v1 = rows_g[b, pl.ds(16, 16)] + qtab_v[pl.ds(qb + 16, 16)]
            bcol = jnp.full((16,), b, jnp.int32)
            plsc.store_scatter(feat_v, [f16a, bcol], v0)
            plsc.store_scatter(feat_v, [f16b, bcol], v1)
            return c2
        lax.fori_loop(0, CB, row, 0)

        cx.wait()
        copies = []
        for ti in range(D // 8):
            copies.append(pltpu.async_copy(
                feat_v.at[pl.ds(ti * 8, 8), pl.ds(0, CB)],
                feat5.at[s, ti, tj], sem_w))
        for ti in range(D // 8):
            copies.append(pltpu.async_copy(
                x_v.at[pl.ds(ti * 8, 8)], feat5.at[s, D // 8 + ti, tj], sem_w))
        for ti in range(D // 8):
            copies.append(pltpu.async_copy(
                x_v.at[pl.ds(ti * 8, 8)], param5.at[s, ti, tj], sem_w))
        for cp in copies:
            cp.wait()
        return carry

    lax.fori_loop(0, n_chunks_per_w, chunk, 0)


def kernel(x, annotators, questions, annotator_embedding, question_embedding):
    B, S, XF = x.shape
    N = B * S
    n_ann = annotator_embedding.shape[0] - 1
    assert B % CB == 0 and N % (NW * CB) == 0
    chunks_per_s = B // CB
    n_chunks_per_w = N // (NW * CB)

    ai = annotators.T.reshape(N).astype(jnp.int32)
    qi = questions.T.reshape(N).astype(jnp.int32)
    qtab = question_embedding.reshape(-1)
    xl = x.transpose(2, 1, 0)

    mesh = plsc.VectorSubcoreMesh(core_axis_name="c", subcore_axis_name="s")
    feat5, param5 = pl.kernel(
        functools.partial(_body, n_chunks_per_w, chunks_per_s, n_ann),
        out_type=(
            jax.ShapeDtypeStruct((S, 2 * D // 8, B // CB, 8, CB), jnp.float32),
            jax.ShapeDtypeStruct((S, D // 8, B // CB, 8, CB), jnp.float32),
        ),
        mesh=mesh,
        compiler_params=pltpu.CompilerParams(
            use_tc_tiling_on_sc=False, needs_layout_passes=False),
        scratch_types=[
            pltpu.VMEM((CB,), jnp.int32),
            pltpu.VMEM((CB + 16,), jnp.int32),
            pltpu.VMEM((question_embedding.size,), jnp.float32),
            pltpu.VMEM((CB, D), jnp.float32),
            pltpu.VMEM((D, FP), jnp.float32),
            pltpu.VMEM((D, CB), jnp.float32),
            pltpu.SemaphoreType.DMA,
            pltpu.SemaphoreType.DMA,
            pltpu.SemaphoreType.DMA,
        ],
    )(annotator_embedding, qtab, ai, qi, xl)

    feature_x = feat5.transpose(2, 4, 0, 1, 3).reshape(B, S, 2 * D)
    param_x = param5.transpose(2, 4, 0, 1, 3).reshape(B, S, D)
    return feature_x, param_x
